# baseline (speedup 1.0000x reference)
import functools

import jax
import jax.numpy as jnp
from jax import lax
from jax.experimental import pallas as pl
from jax.experimental.pallas import tpu as pltpu
from jax.experimental.pallas import tpu_sc as plsc

MAX_LEN = 8192
HIDDEN_DIM = 1024

_INFO = plsc.get_sparse_core_info()
_NC = _INFO.num_cores
_NS = _INFO.num_subcores
_NW = _NC * _NS

_K = 2560
_B_PER_W = _K // _NW
_CHUNK = 16
_NCHUNK = _B_PER_W // _CHUNK
_NBUF = 3

_BR = 256
_TC_ROWS = MAX_LEN - _K


def _sc_copy(table_hbm, out_hbm, rows_v, *sems):
    gsems, ssems = sems[:_NBUF], sems[_NBUF:]
    wid = lax.axis_index("s") * _NC + lax.axis_index("c")
    base = wid * _B_PER_W

    def gather(c):
        return pltpu.async_copy(
            table_hbm.at[pl.ds(base + c * _CHUNK, _CHUNK)],
            rows_v.at[c % _NBUF], gsems[c % _NBUF])

    gh = [None] * _NCHUNK
    sh = [None] * _NCHUNK
    for c in range(min(_NBUF, _NCHUNK)):
        gh[c] = gather(c)
    for c in range(_NCHUNK):
        gh[c].wait()
        sh[c] = pltpu.async_copy(
            rows_v.at[c % _NBUF],
            out_hbm.at[pl.ds(base + c * _CHUNK, _CHUNK)], ssems[c % _NBUF])
        if c + _NBUF < _NCHUNK:
            sh[c].wait()
            gh[c + _NBUF] = gather(c + _NBUF)
    for c in range(max(0, _NCHUNK - _NBUF), _NCHUNK):
        sh[c].wait()


def _tc_body(in_ref, out_ref):
    out_ref[...] = in_ref[...]


def kernel(seq_len, pos_embedding):
    del seq_len
    sc_kern = functools.partial(
        pl.kernel,
        mesh=plsc.VectorSubcoreMesh(core_axis_name="c", subcore_axis_name="s"),
        out_type=jax.ShapeDtypeStruct((_K, HIDDEN_DIM), jnp.float32),
        scratch_types=[
            pltpu.VMEM((_NBUF, _CHUNK, HIDDEN_DIM), jnp.float32),
        ] + [pltpu.SemaphoreType.DMA] * (2 * _NBUF),
    )(_sc_copy)
    top = sc_kern(pos_embedding)
    bot = pl.pallas_call(
        _tc_body,
        grid=(_TC_ROWS // _BR,),
        in_specs=[pl.BlockSpec((_BR, HIDDEN_DIM),
                               lambda i: (i + _K // _BR, 0))],
        out_specs=pl.BlockSpec((_BR, HIDDEN_DIM), lambda i: (i, 0)),
        out_shape=jax.ShapeDtypeStruct((_TC_ROWS, HIDDEN_DIM), jnp.float32),
    )(pos_embedding)
    return jnp.concatenate([top, bot], axis=0)


# speedup vs baseline: 1.4545x; 1.4545x over previous
import functools

import jax
import jax.numpy as jnp
from jax import lax
from jax.experimental import pallas as pl
from jax.experimental.pallas import tpu as pltpu
from jax.experimental.pallas import tpu_sc as plsc

MAX_LEN = 8192
HIDDEN_DIM = 1024

_INFO = plsc.get_sparse_core_info()
_NC = _INFO.num_cores
_NS = _INFO.num_subcores
_L = _INFO.num_lanes
_NW = _NC * _NS
_B_PER_W = MAX_LEN // _NW
_CHUNK = 32
_NCHUNK = _B_PER_W // _CHUNK
_NBUF = 3


def _pos_encoding_kernel(slen_hbm, table_hbm, out_hbm, slen_v, idx_v,
                         rows_v, *sems):
    gsems, ssems = sems[:_NBUF], sems[_NBUF:]
    wid = lax.axis_index("s") * _NC + lax.axis_index("c")
    base = wid * _B_PER_W

    pltpu.sync_copy(slen_hbm, slen_v)
    lim = slen_v[...]

    for c in range(_NCHUNK):
        for j in range(_CHUNK // _L):
            vec = lax.iota(jnp.int32, _L) + (base + c * _CHUNK + j * _L)
            idx_v[c, pl.ds(j * _L, _L)] = jnp.minimum(vec, lim)

    def gather(c):
        return pltpu.async_copy(table_hbm.at[idx_v.at[c]],
                                rows_v.at[c % _NBUF], gsems[c % _NBUF])

    gh = [None] * _NCHUNK
    sh = [None] * _NCHUNK
    for c in range(_NBUF):
        gh[c] = gather(c)
    for c in range(_NCHUNK):
        gh[c].wait()
        sh[c] = pltpu.async_copy(
            rows_v.at[c % _NBUF],
            out_hbm.at[pl.ds(base + c * _CHUNK, _CHUNK)], ssems[c % _NBUF])
        prev = c - 1
        if prev >= 0 and prev + _NBUF < _NCHUNK:
            sh[prev].wait()
            gh[prev + _NBUF] = gather(prev + _NBUF)
    for c in range(_NCHUNK - _NBUF, _NCHUNK):
        sh[c].wait()


def kernel(seq_len, pos_embedding):
    slen = jnp.full((_L,), jnp.asarray(seq_len, jnp.int32) - 1, jnp.int32)
    kern = functools.partial(
        pl.kernel,
        mesh=plsc.VectorSubcoreMesh(core_axis_name="c", subcore_axis_name="s"),
        out_type=jax.ShapeDtypeStruct((MAX_LEN, HIDDEN_DIM), jnp.float32),
        scratch_types=[
            pltpu.VMEM((_L,), jnp.int32),
            pltpu.VMEM((_NCHUNK, _CHUNK), jnp.int32),
            pltpu.VMEM((_NBUF, _CHUNK, HIDDEN_DIM), jnp.float32),
        ] + [pltpu.SemaphoreType.DMA] * (2 * _NBUF),
    )(_pos_encoding_kernel)
    return kern(slen, pos_embedding)


# speedup vs baseline: 1.5457x; 1.0627x over previous
import functools

import jax
import jax.numpy as jnp
from jax import lax
from jax.experimental import pallas as pl
from jax.experimental.pallas import tpu as pltpu
from jax.experimental.pallas import tpu_sc as plsc

MAX_LEN = 8192
HIDDEN_DIM = 1024

_INFO = plsc.get_sparse_core_info()
_NC = _INFO.num_cores
_NS = _INFO.num_subcores
_NW = _NC * _NS
_B_PER_W = MAX_LEN // _NW
_CHUNK = 32
_NCHUNK = _B_PER_W // _CHUNK
_NBUF = 3


def _pos_encoding_kernel(pos_hbm, table_hbm, out_hbm, idx_v, rows_v, *sems):
    gsems, ssems = sems[:_NBUF], sems[_NBUF:]
    wid = lax.axis_index("s") * _NC + lax.axis_index("c")
    base = wid * _B_PER_W

    pltpu.sync_copy(pos_hbm.at[wid], idx_v)

    def gather(c):
        return pltpu.async_copy(table_hbm.at[idx_v.at[c]],
                                rows_v.at[c % _NBUF], gsems[c % _NBUF])

    gh = [None] * _NCHUNK
    sh = [None] * _NCHUNK
    for c in range(_NBUF):
        gh[c] = gather(c)
    for c in range(_NCHUNK):
        gh[c].wait()
        sh[c] = pltpu.async_copy(
            rows_v.at[c % _NBUF],
            out_hbm.at[pl.ds(base + c * _CHUNK, _CHUNK)], ssems[c % _NBUF])
        if c + _NBUF < _NCHUNK:
            sh[c].wait()
            gh[c + _NBUF] = gather(c + _NBUF)
    for c in range(_NCHUNK - _NBUF, _NCHUNK):
        sh[c].wait()


def kernel(seq_len, pos_embedding):
    positions = jnp.minimum(
        jnp.arange(MAX_LEN, dtype=jnp.int32),
        jnp.asarray(seq_len, jnp.int32) - 1,
    ).reshape(_NW, _NCHUNK, _CHUNK)
    kern = functools.partial(
        pl.kernel,
        mesh=plsc.VectorSubcoreMesh(core_axis_name="c", subcore_axis_name="s"),
        out_type=jax.ShapeDtypeStruct((MAX_LEN, HIDDEN_DIM), jnp.float32),
        scratch_types=[
            pltpu.VMEM((_NCHUNK, _CHUNK), jnp.int32),
            pltpu.VMEM((_NBUF, _CHUNK, HIDDEN_DIM), jnp.float32),
        ] + [pltpu.SemaphoreType.DMA] * (2 * _NBUF),
    )(_pos_encoding_kernel)
    return kern(positions, pos_embedding)


# speedup vs baseline: 1.5481x; 1.0016x over previous
import functools

import jax
import jax.numpy as jnp
from jax import lax
from jax.experimental import pallas as pl
from jax.experimental.pallas import tpu as pltpu
from jax.experimental.pallas import tpu_sc as plsc

MAX_LEN = 8192
HIDDEN_DIM = 1024

_INFO = plsc.get_sparse_core_info()
_NC = _INFO.num_cores
_NS = _INFO.num_subcores
_NW = _NC * _NS
_B_PER_W = MAX_LEN // _NW
_CHUNK = 16
_NCHUNK = _B_PER_W // _CHUNK
_NBUF = 6


def _pos_encoding_kernel(pos_hbm, table_hbm, out_hbm, idx_v, rows_v, *sems):
    gsems, ssems = sems[:_NBUF], sems[_NBUF:]
    wid = lax.axis_index("s") * _NC + lax.axis_index("c")
    base = wid * _B_PER_W

    pltpu.sync_copy(pos_hbm.at[wid], idx_v)

    def gather(c):
        return pltpu.async_copy(table_hbm.at[idx_v.at[c]],
                                rows_v.at[c % _NBUF], gsems[c % _NBUF])

    gh = [None] * _NCHUNK
    sh = [None] * _NCHUNK
    for c in range(_NBUF):
        gh[c] = gather(c)
    for c in range(_NCHUNK):
        gh[c].wait()
        sh[c] = pltpu.async_copy(
            rows_v.at[c % _NBUF],
            out_hbm.at[pl.ds(base + c * _CHUNK, _CHUNK)], ssems[c % _NBUF])
        if c + _NBUF < _NCHUNK:
            sh[c].wait()
            gh[c + _NBUF] = gather(c + _NBUF)
    for c in range(_NCHUNK - _NBUF, _NCHUNK):
        sh[c].wait()


def kernel(seq_len, pos_embedding):
    positions = jnp.minimum(
        jnp.arange(MAX_LEN, dtype=jnp.int32),
        jnp.asarray(seq_len, jnp.int32) - 1,
    ).reshape(_NW, _NCHUNK, _CHUNK)
    kern = functools.partial(
        pl.kernel,
        mesh=plsc.VectorSubcoreMesh(core_axis_name="c", subcore_axis_name="s"),
        out_type=jax.ShapeDtypeStruct((MAX_LEN, HIDDEN_DIM), jnp.float32),
        scratch_types=[
            pltpu.VMEM((_NCHUNK, _CHUNK), jnp.int32),
            pltpu.VMEM((_NBUF, _CHUNK, HIDDEN_DIM), jnp.float32),
        ] + [pltpu.SemaphoreType.DMA] * (2 * _NBUF),
    )(_pos_encoding_kernel)
    return kern(positions, pos_embedding)
